# baseline (speedup 1.0000x reference)
import functools

import jax
import jax.numpy as jnp
from jax import lax
from jax.experimental import pallas as pl
from jax.experimental.pallas import tpu as pltpu
from jax.experimental.pallas import tpu_sc as plsc

_NUM_HEADS = 12
_HEAD_DIM = 64
_MAX_REL = 128
_SEQ = 512
_VOCAB = 2 * _MAX_REL + 1
_PAD_L = _SEQ - 1 - _MAX_REL
_PW = 1024
_LANES = 16
_NSHIFT = 8


def _sc_rel_pos(table):
    info = plsc.get_sparse_core_info()
    num_cores = info.num_cores
    num_subcores = info.num_subcores
    nw = num_cores * num_subcores
    pairs = _NUM_HEADS * _SEQ
    per = pairs // nw
    assert per * nw == pairs
    p8_rows = _NSHIFT * _HEAD_DIM
    rows_per_tile = p8_rows // num_subcores

    mesh = plsc.VectorSubcoreMesh(core_axis_name="c", subcore_axis_name="s")

    @functools.partial(
        pl.kernel,
        mesh=mesh,
        out_type=(
            jax.ShapeDtypeStruct((_NUM_HEADS, _HEAD_DIM, _SEQ, _SEQ),
                                 jnp.float32),
            jax.ShapeDtypeStruct((2, p8_rows, _PW), jnp.float32),
        ),
        scratch_types=[
            pltpu.VMEM((_VOCAB, _HEAD_DIM), jnp.float32),
            pltpu.VMEM((rows_per_tile, _PW), jnp.float32),
            pltpu.SemaphoreType.DMA,
            pltpu.SemaphoreType.DMA,
        ],
        compiler_params=pltpu.CompilerParams(
            use_tc_tiling_on_sc=False, needs_layout_passes=False),
    )
    def k(table_hbm, out_hbm, p8_hbm, table_v, build_v, sem_in, sem_out):
        sid = lax.axis_index("s")
        cid = lax.axis_index("c")
        wid = sid * num_cores + cid

        pltpu.async_copy(table_hbm, table_v, sem_in).wait()

        lane = lax.iota(jnp.int32, _LANES)

        def build_row(rr, carry):
            rd = sid * rows_per_tile + rr
            r = rd // _HEAD_DIM
            d = rd - r * _HEAD_DIM
            dv = jnp.full((_LANES,), d, jnp.int32)

            def build_chunk(c, inner):
                q = c * _LANES + lane
                pos = jnp.clip(q + r - _PAD_L, 0, _VOCAB - 1)
                vals = plsc.load_gather(table_v, [pos, dv])
                build_v[rr, pl.ds(c * _LANES, _LANES)] = vals
                return inner

            return lax.fori_loop(0, _PW // _LANES, build_chunk, carry)

        lax.fori_loop(0, rows_per_tile, build_row, 0)

        pltpu.sync_copy(
            build_v,
            p8_hbm.at[cid, pl.ds(sid * rows_per_tile, rows_per_tile), :])
        plsc.subcore_barrier()

        def fire(t, carry):
            pair = wid * per + t
            h = pair // _SEQ
            i = pair - h * _SEQ
            start = (_SEQ - 1) - i
            r = lax.rem(start, _NSHIFT)
            astart = pl.multiple_of(start - r, _NSHIFT)
            src = p8_hbm.at[cid, pl.ds(r * _HEAD_DIM, _HEAD_DIM),
                            pl.ds(astart, _SEQ)]
            dst = out_hbm.at[h, :, i, :]
            pltpu.make_async_copy(src, dst, sem_out).start()
            return carry

        lax.fori_loop(0, per, fire, 0)

        def drain(t, carry):
            pltpu.make_async_copy(
                p8_hbm.at[cid, pl.ds(0, _HEAD_DIM), pl.ds(0, _SEQ)],
                out_hbm.at[0, :, 0, :],
                sem_out,
            ).wait()
            return carry

        lax.fori_loop(0, per, drain, 0)

    return k(table)[0]


def kernel(table, seq_len):
    del seq_len
    return _sc_rel_pos(table)


# speedup vs baseline: 11.2552x; 11.2552x over previous
import functools

import jax
import jax.numpy as jnp
from jax import lax
from jax.experimental import pallas as pl
from jax.experimental.pallas import tpu as pltpu
from jax.experimental.pallas import tpu_sc as plsc

_NUM_HEADS = 12
_HEAD_DIM = 64
_MAX_REL = 128
_SEQ = 512
_VOCAB = 2 * _MAX_REL + 1
_PAD_L = _SEQ - 1 - _MAX_REL
_PW = 1024
_LANES = 16
_NSUB = 8
_PS_ROWS = _HEAD_DIM * _NSUB


def _sc_build_ps(table):
    info = plsc.get_sparse_core_info()
    num_cores = info.num_cores
    nw = num_cores * info.num_subcores
    rows_per_tile = _PS_ROWS // nw

    mesh = plsc.VectorSubcoreMesh(core_axis_name="c", subcore_axis_name="s")

    @functools.partial(
        pl.kernel,
        mesh=mesh,
        out_type=jax.ShapeDtypeStruct((_PS_ROWS, _PW), jnp.float32),
        scratch_types=[
            pltpu.VMEM((_VOCAB, _HEAD_DIM), jnp.float32),
            pltpu.VMEM((rows_per_tile, _PW), jnp.float32),
            pltpu.SemaphoreType.DMA,
        ],
        compiler_params=pltpu.CompilerParams(
            use_tc_tiling_on_sc=False, needs_layout_passes=False),
    )
    def k(table_hbm, ps_hbm, table_v, build_v, sem_in):
        wid = lax.axis_index("s") * num_cores + lax.axis_index("c")

        pltpu.async_copy(table_hbm, table_v, sem_in).wait()

        lane = lax.iota(jnp.int32, _LANES)

        def build_row(rr, carry):
            rd = wid * rows_per_tile + rr
            d = rd // _NSUB
            s = rd - d * _NSUB
            dv = jnp.full((_LANES,), d, jnp.int32)

            def build_chunk(c, inner):
                q = c * _LANES + lane
                pos = jnp.clip(q - s - _PAD_L, 0, _VOCAB - 1)
                vals = plsc.load_gather(table_v, [pos, dv])
                build_v[rr, pl.ds(c * _LANES, _LANES)] = vals
                return inner

            return lax.fori_loop(0, _PW // _LANES, build_chunk, carry)

        lax.fori_loop(0, rows_per_tile, build_row, 0)

        pltpu.sync_copy(
            build_v,
            ps_hbm.at[pl.ds(wid * rows_per_tile, rows_per_tile), :])

    return k(table)


def _tc_materialize(ps):
    n_iblk = _SEQ // _HEAD_DIM

    def body(ps_ref, o_ref):
        i0 = pl.program_id(2) * _HEAD_DIM
        rows = ps_ref[:, :]
        for sg in range(_HEAD_DIM // _NSUB):
            start = (_SEQ - 1) - i0 - sg * _NSUB
            w = pltpu.roll(rows, -start, axis=1)
            o_ref[0, 0, pl.ds(sg * _NSUB, _NSUB), :] = w[:, : _SEQ]

    return pl.pallas_call(
        body,
        grid=(_NUM_HEADS, _HEAD_DIM, n_iblk),
        in_specs=[
            pl.BlockSpec((_NSUB, _PW), lambda h, d, ib: (d, 0)),
        ],
        out_specs=pl.BlockSpec(
            (1, 1, _HEAD_DIM, _SEQ), lambda h, d, ib: (h, d, ib, 0)),
        out_shape=jax.ShapeDtypeStruct(
            (_NUM_HEADS, _HEAD_DIM, _SEQ, _SEQ), jnp.float32),
        compiler_params=pltpu.CompilerParams(
            dimension_semantics=("parallel", "arbitrary", "arbitrary")),
    )(ps)


def kernel(table, seq_len):
    del seq_len
    ps = _sc_build_ps(table)
    return _tc_materialize(ps)


# speedup vs baseline: 13.7810x; 1.2244x over previous
import functools

import jax
import jax.numpy as jnp
from jax import lax
from jax.experimental import pallas as pl
from jax.experimental.pallas import tpu as pltpu
from jax.experimental.pallas import tpu_sc as plsc

_NUM_HEADS = 12
_HEAD_DIM = 64
_MAX_REL = 128
_SEQ = 512
_VOCAB = 2 * _MAX_REL + 1
_PAD_L = _SEQ - 1 - _MAX_REL
_PW = 1024
_LANES = 16
_NSUB = 8
_PS_ROWS = _HEAD_DIM * _NSUB


def _sc_build_ps(table):
    info = plsc.get_sparse_core_info()
    num_cores = info.num_cores
    nw = num_cores * info.num_subcores
    rows_per_tile = _PS_ROWS // nw

    mesh = plsc.VectorSubcoreMesh(core_axis_name="c", subcore_axis_name="s")

    @functools.partial(
        pl.kernel,
        mesh=mesh,
        out_type=jax.ShapeDtypeStruct((_PS_ROWS, _PW), jnp.float32),
        scratch_types=[
            pltpu.VMEM((_VOCAB, _HEAD_DIM), jnp.float32),
            pltpu.VMEM((rows_per_tile, _PW), jnp.float32),
            pltpu.SemaphoreType.DMA,
        ],
        compiler_params=pltpu.CompilerParams(
            use_tc_tiling_on_sc=False, needs_layout_passes=False),
    )
    def k(table_hbm, ps_hbm, table_v, build_v, sem_in):
        wid = lax.axis_index("s") * num_cores + lax.axis_index("c")

        pltpu.async_copy(table_hbm, table_v, sem_in).wait()

        lane = lax.iota(jnp.int32, _LANES)

        def build_row(rr, carry):
            rd = wid * rows_per_tile + rr
            d = rd // _NSUB
            s = rd - d * _NSUB
            dv = jnp.full((_LANES,), d, jnp.int32)

            def build_chunk(c, inner):
                q = c * _LANES + lane
                pos = jnp.clip(q - s - (_PAD_L + 1), 0, _VOCAB - 1)
                vals = plsc.load_gather(table_v, [pos, dv])
                build_v[rr, pl.ds(c * _LANES, _LANES)] = vals
                return inner

            return lax.fori_loop(0, _PW // _LANES, build_chunk, carry)

        lax.fori_loop(0, rows_per_tile, build_row, 0)

        pltpu.sync_copy(
            build_v,
            ps_hbm.at[pl.ds(wid * rows_per_tile, rows_per_tile), :])

    return k(table)


def _tc_materialize(ps):
    n_iblk = _SEQ // _HEAD_DIM

    def body(ps_ref, o_ref, scr):
        h = pl.program_id(1)
        ib = pl.program_id(2)

        @pl.when(jnp.logical_and(h == 0, ib == 0))
        def _prologue():
            rows = ps_ref[:, :]
            scr[0] = rows
            for c in range(1, 16):
                scr[c] = pltpu.roll(rows, _PW - _NSUB * c, axis=1)

        odd = lax.rem(ib, 2)

        for sg in range(_HEAD_DIM // _NSUB):
            c_e = 0 if sg == 0 else 16 - sg
            c_o = 8 - sg

            def _store(c, lq_base, sg=sg):
                lq = pl.multiple_of(lq_base - _HEAD_DIM * ib, 128)
                o_ref[0, 0, pl.ds(sg * _NSUB, _NSUB), :] = (
                    scr[c, :, pl.ds(lq, _SEQ)])

            pl.when(odd == 0)(
                functools.partial(_store, c_e, 512 if sg == 0 else 384))
            pl.when(odd == 1)(functools.partial(_store, c_o, 448))

    return pl.pallas_call(
        body,
        grid=(_HEAD_DIM, _NUM_HEADS, n_iblk),
        in_specs=[
            pl.BlockSpec((_NSUB, _PW), lambda d, h, ib: (d, 0)),
        ],
        out_specs=pl.BlockSpec(
            (1, 1, _HEAD_DIM, _SEQ), lambda d, h, ib: (h, d, ib, 0)),
        out_shape=jax.ShapeDtypeStruct(
            (_NUM_HEADS, _HEAD_DIM, _SEQ, _SEQ), jnp.float32),
        scratch_shapes=[pltpu.VMEM((16, _NSUB, _PW), jnp.float32)],
        compiler_params=pltpu.CompilerParams(
            dimension_semantics=("arbitrary", "arbitrary", "arbitrary")),
    )(ps)


def kernel(table, seq_len):
    del seq_len
    ps = _sc_build_ps(table)
    return _tc_materialize(ps)


# speedup vs baseline: 62.6866x; 4.5488x over previous
import functools

import jax
import jax.numpy as jnp
from jax import lax
from jax.experimental import pallas as pl
from jax.experimental.pallas import tpu as pltpu
from jax.experimental.pallas import tpu_sc as plsc

_NUM_HEADS = 12
_HEAD_DIM = 64
_MAX_REL = 128
_SEQ = 512
_VOCAB = 2 * _MAX_REL + 1
_PAD_L = _SEQ - 1 - _MAX_REL
_PW = 1024
_LANES = 16
_NSUB = 8
_PS_ROWS = _HEAD_DIM * _NSUB


def _sc_build_ps(table):
    info = plsc.get_sparse_core_info()
    num_cores = info.num_cores
    nw = num_cores * info.num_subcores
    rows_per_tile = _PS_ROWS // nw

    mesh = plsc.VectorSubcoreMesh(core_axis_name="c", subcore_axis_name="s")

    @functools.partial(
        pl.kernel,
        mesh=mesh,
        out_type=jax.ShapeDtypeStruct((_PS_ROWS, _PW), jnp.float32),
        scratch_types=[
            pltpu.VMEM((_VOCAB, _HEAD_DIM), jnp.float32),
            pltpu.VMEM((rows_per_tile, _PW), jnp.float32),
            pltpu.SemaphoreType.DMA,
        ],
        compiler_params=pltpu.CompilerParams(
            use_tc_tiling_on_sc=False, needs_layout_passes=False),
    )
    def k(table_hbm, ps_hbm, table_v, build_v, sem_in):
        wid = lax.axis_index("s") * num_cores + lax.axis_index("c")

        pltpu.async_copy(table_hbm, table_v, sem_in).wait()

        lane = lax.iota(jnp.int32, _LANES)

        def build_row(rr, carry):
            rd = wid * rows_per_tile + rr
            d = rd // _NSUB
            s = rd - d * _NSUB
            dv = jnp.full((_LANES,), d, jnp.int32)

            def build_chunk(c, inner):
                q = c * _LANES + lane
                pos = jnp.clip(q - s - (_PAD_L + 1), 0, _VOCAB - 1)
                vals = plsc.load_gather(table_v, [pos, dv])
                build_v[rr, pl.ds(c * _LANES, _LANES)] = vals
                return inner

            return lax.fori_loop(0, _PW // _LANES, build_chunk, carry)

        lax.fori_loop(0, rows_per_tile, build_row, 0)

        pltpu.sync_copy(
            build_v,
            ps_hbm.at[pl.ds(wid * rows_per_tile, rows_per_tile), :])

    return k(table)


def _tc_materialize(ps):
    def body(ps_ref, o_ref, scr):
        h = pl.program_id(1)

        @pl.when(h == 0)
        def _prologue():
            rows = ps_ref[:, :]
            scr[0] = rows
            for c in range(1, 16):
                scr[c] = pltpu.roll(rows, _PW - _NSUB * c, axis=1)

        for ib in range(_SEQ // _HEAD_DIM):
            for sg in range(_HEAD_DIM // _NSUB):
                if ib % 2 == 0:
                    c = 0 if sg == 0 else 16 - sg
                    lq = (512 if sg == 0 else 384) - _HEAD_DIM * ib
                else:
                    c = 8 - sg
                    lq = 448 - _HEAD_DIM * ib
                i0 = ib * _HEAD_DIM + sg * _NSUB
                o_ref[0, 0, pl.ds(i0, _NSUB), :] = (
                    scr[c, :, pl.ds(lq, _SEQ)])

    return pl.pallas_call(
        body,
        grid=(_HEAD_DIM, _NUM_HEADS),
        in_specs=[
            pl.BlockSpec((_NSUB, _PW), lambda d, h: (d, 0)),
        ],
        out_specs=pl.BlockSpec(
            (1, 1, _SEQ, _SEQ), lambda d, h: (h, d, 0, 0)),
        out_shape=jax.ShapeDtypeStruct(
            (_NUM_HEADS, _HEAD_DIM, _SEQ, _SEQ), jnp.float32),
        scratch_shapes=[pltpu.VMEM((16, _NSUB, _PW), jnp.float32)],
        compiler_params=pltpu.CompilerParams(
            dimension_semantics=("arbitrary", "arbitrary")),
    )(ps)


def kernel(table, seq_len):
    del seq_len
    ps = _sc_build_ps(table)
    return _tc_materialize(ps)


# speedup vs baseline: 66.4817x; 1.0605x over previous
import functools

import jax
import jax.numpy as jnp
from jax import lax
from jax.experimental import pallas as pl
from jax.experimental.pallas import tpu as pltpu
from jax.experimental.pallas import tpu_sc as plsc

_NUM_HEADS = 12
_HEAD_DIM = 64
_MAX_REL = 128
_SEQ = 512
_VOCAB = 2 * _MAX_REL + 1
_PAD_L = _SEQ - 1 - _MAX_REL
_PW = 1024
_LANES = 16
_NSUB = 8
_PS_ROWS = _HEAD_DIM * _NSUB


def _sc_build_ps(table):
    info = plsc.get_sparse_core_info()
    num_cores = info.num_cores
    nw = num_cores * info.num_subcores
    rows_per_tile = _PS_ROWS // nw

    mesh = plsc.VectorSubcoreMesh(core_axis_name="c", subcore_axis_name="s")

    @functools.partial(
        pl.kernel,
        mesh=mesh,
        out_type=jax.ShapeDtypeStruct((_PS_ROWS, _PW), jnp.float32),
        scratch_types=[
            pltpu.VMEM((_VOCAB, _HEAD_DIM), jnp.float32),
            pltpu.VMEM((rows_per_tile, _PW), jnp.float32),
            pltpu.SemaphoreType.DMA,
        ],
        compiler_params=pltpu.CompilerParams(
            use_tc_tiling_on_sc=False, needs_layout_passes=False),
    )
    def k(table_hbm, ps_hbm, table_v, build_v, sem_in):
        wid = lax.axis_index("s") * num_cores + lax.axis_index("c")

        pltpu.async_copy(table_hbm, table_v, sem_in).wait()

        lane = lax.iota(jnp.int32, _LANES)

        def build_row(rr, carry):
            rd = wid * rows_per_tile + rr
            d = rd // _NSUB
            s = rd - d * _NSUB
            dv = jnp.full((_LANES,), d, jnp.int32)

            def build_chunk(c, inner):
                q = c * _LANES + lane
                pos = jnp.clip(q - s - (_PAD_L + 1), 0, _VOCAB - 1)
                vals = plsc.load_gather(table_v, [pos, dv])
                build_v[rr, pl.ds(c * _LANES, _LANES)] = vals
                return inner

            return lax.fori_loop(0, _PW // _LANES, build_chunk, carry)

        lax.fori_loop(0, rows_per_tile, build_row, 0)

        pltpu.sync_copy(
            build_v,
            ps_hbm.at[pl.ds(wid * rows_per_tile, rows_per_tile), :])

    return k(table)


def _tc_materialize(ps):
    def body(ps_ref, o_ref, scr):
        h = pl.program_id(0)
        d = pl.program_id(1)

        @pl.when(jnp.logical_and(h == 0, d == 0))
        def _prologue():
            for dd in range(_HEAD_DIM):
                rows = ps_ref[pl.ds(dd * _NSUB, _NSUB), :]
                scr[dd, 0] = rows
                for c in range(1, 16):
                    scr[dd, c] = pltpu.roll(rows, _PW - _NSUB * c, axis=1)

        for ib in range(_SEQ // _HEAD_DIM):
            for sg in range(_HEAD_DIM // _NSUB):
                if ib % 2 == 0:
                    c = 0 if sg == 0 else 16 - sg
                    lq = (512 if sg == 0 else 384) - _HEAD_DIM * ib
                else:
                    c = 8 - sg
                    lq = 448 - _HEAD_DIM * ib
                i0 = ib * _HEAD_DIM + sg * _NSUB
                o_ref[0, 0, pl.ds(i0, _NSUB), :] = (
                    scr[d, c, :, pl.ds(lq, _SEQ)])

    return pl.pallas_call(
        body,
        grid=(_NUM_HEADS, _HEAD_DIM),
        in_specs=[
            pl.BlockSpec((_PS_ROWS, _PW), lambda h, d: (0, 0)),
        ],
        out_specs=pl.BlockSpec(
            (1, 1, _SEQ, _SEQ), lambda h, d: (h, d, 0, 0)),
        out_shape=jax.ShapeDtypeStruct(
            (_NUM_HEADS, _HEAD_DIM, _SEQ, _SEQ), jnp.float32),
        scratch_shapes=[
            pltpu.VMEM((_HEAD_DIM, 16, _NSUB, _PW), jnp.float32)],
        compiler_params=pltpu.CompilerParams(
            dimension_semantics=("arbitrary", "arbitrary")),
    )(ps)


def kernel(table, seq_len):
    del seq_len
    ps = _sc_build_ps(table)
    return _tc_materialize(ps)
